# baseline (speedup 1.0000x reference)
import jax
import jax.numpy as jnp
from jax.experimental import pallas as pl
from jax.experimental.pallas import tpu as pltpu

D = 2048
F = 4096
E = 8
T = 2048
BT = 256
MAXB = T // BT + E - 1
PADN = MAXB * BT
BJ = 256
NJ = F // BJ


def _router_body(x_ref, rw_ref, rb_ref, tw_ref, ti_ref):
    l = jnp.dot(x_ref[...], rw_ref[...], preferred_element_type=jnp.float32)
    l = l + rb_ref[...]
    m = jnp.max(l, axis=1, keepdims=True)
    s = jnp.sum(jnp.exp(l - m), axis=1, keepdims=True)
    tw_ref[...] = 1.0 / s
    iota = jax.lax.broadcasted_iota(jnp.int32, l.shape, 1)
    ti_ref[...] = jnp.min(jnp.where(l >= m, iota, E), axis=1, keepdims=True)


def _router(flat, rw, rb):
    return pl.pallas_call(
        _router_body,
        out_shape=(
            jax.ShapeDtypeStruct((T, 1), jnp.float32),
            jax.ShapeDtypeStruct((T, 1), jnp.int32),
        ),
    )(flat, rw, rb.reshape(1, E))


def _moe_body(sp_ref, x_ref, wg_ref, wu_ref, wd_ref, tw_ref, o_ref):
    e = pl.program_id(0)
    j = pl.program_id(1)
    row0 = sp_ref[e]
    nblk = sp_ref[E + e]

    def blk(k, _):
        rows = pl.ds(pl.multiple_of(row0 + k * BT, BT), BT)
        x = x_ref[rows, :]
        g = jnp.dot(x, wg_ref[0], preferred_element_type=jnp.float32)
        u = jnp.dot(x, wu_ref[0], preferred_element_type=jnp.float32)
        h = (jax.nn.silu(g) * u).astype(jnp.bfloat16)
        part = jnp.dot(h, wd_ref[0], preferred_element_type=jnp.float32)

        @pl.when(j == 0)
        def _():
            o_ref[rows, :] = part

        @pl.when(j > 0)
        def _():
            o_ref[rows, :] = o_ref[rows, :] + part

        @pl.when(j == NJ - 1)
        def _():
            o_ref[rows, :] = o_ref[rows, :] * tw_ref[rows, :]

        return 0

    jax.lax.fori_loop(0, nblk, blk, 0)


def _grouped_mlp(x_p, Wg, Wu, Wd, tw_p, sp):
    grid_spec = pltpu.PrefetchScalarGridSpec(
        num_scalar_prefetch=1,
        grid=(E, NJ),
        in_specs=[
            pl.BlockSpec((PADN, D), lambda e, j, sp: (0, 0)),
            pl.BlockSpec((1, D, BJ), lambda e, j, sp: (e, 0, j)),
            pl.BlockSpec((1, D, BJ), lambda e, j, sp: (e, 0, j)),
            pl.BlockSpec((1, BJ, D), lambda e, j, sp: (e, j, 0)),
            pl.BlockSpec((PADN, 1), lambda e, j, sp: (0, 0)),
        ],
        out_specs=pl.BlockSpec((PADN, D), lambda e, j, sp: (0, 0)),
    )
    return pl.pallas_call(
        _moe_body,
        grid_spec=grid_spec,
        out_shape=jax.ShapeDtypeStruct((PADN, D), jnp.float32),
    )(sp, x_p, Wg, Wu, Wd, tw_p)


def kernel(hidden_states, router_W, router_b, Wg, Wu, Wd):
    B, S, _ = hidden_states.shape
    flat = hidden_states.reshape(T, D)

    tw, ti = _router(flat, router_W, router_b)
    topi = ti[:, 0]
    topw = tw[:, 0]

    order = jnp.argsort(topi, stable=True).astype(jnp.int32)
    counts = jnp.bincount(topi, length=E)
    nb = (counts + BT - 1) // BT
    cum_nb = jnp.cumsum(nb)
    pstart = (cum_nb - nb) * BT
    cstart = jnp.cumsum(counts) - counts
    e_sorted = topi[order]
    pos = (pstart[e_sorted] + jnp.arange(T) - cstart[e_sorted]).astype(jnp.int32)
    dest = jnp.zeros((T,), jnp.int32).at[order].set(pos)
    src = jnp.zeros((PADN,), jnp.int32).at[pos].set(order)
    sp = jnp.concatenate([pstart, nb]).astype(jnp.int32)

    x_p = flat.astype(jnp.bfloat16)[src]
    tw_p = topw[src].reshape(PADN, 1)

    y_p = _grouped_mlp(x_p, Wg.astype(jnp.bfloat16), Wu.astype(jnp.bfloat16),
                       Wd.astype(jnp.bfloat16), tw_p, sp)
    out = y_p[dest]
    return out.reshape(B, S, D)


# speedup vs baseline: 1.0997x; 1.0997x over previous
import jax
import jax.numpy as jnp
from jax.experimental import pallas as pl
from jax.experimental.pallas import tpu as pltpu

D = 2048
F = 4096
E = 8
T = 2048
BT = 256
PADX = T + 8 * E + BT
BJ = 512
NJ = F // BJ


def _router_body(x_ref, rw_ref, rb_ref, tw_ref, ti_ref):
    l = jnp.dot(x_ref[...], rw_ref[...], preferred_element_type=jnp.float32)
    l = l + rb_ref[...]
    m = jnp.max(l, axis=1, keepdims=True)
    s = jnp.sum(jnp.exp(l - m), axis=1, keepdims=True)
    tw_ref[...] = 1.0 / s
    iota = jax.lax.broadcasted_iota(jnp.int32, l.shape, 1)
    ti_ref[...] = jnp.min(jnp.where(l >= m, iota, E), axis=1, keepdims=True)


def _router(flat, rw, rb):
    return pl.pallas_call(
        _router_body,
        out_shape=(
            jax.ShapeDtypeStruct((T, 1), jnp.float32),
            jax.ShapeDtypeStruct((T, 1), jnp.int32),
        ),
    )(flat, rw, rb.reshape(1, E))


def _moe_body(sp_ref, x_ref, wg_ref, wu_ref, wd_ref, tw_ref, o_ref):
    j = pl.program_id(1)
    e = pl.program_id(0)
    row0 = sp_ref[e]
    nblk = sp_ref[E + e]

    def blk(k, _):
        rows = pl.ds(pl.multiple_of(row0 + k * BT, 8), BT)
        x = x_ref[rows, :]
        g = jnp.dot(x, wg_ref[0], preferred_element_type=jnp.float32)
        u = jnp.dot(x, wu_ref[0], preferred_element_type=jnp.float32)
        h = (jax.nn.silu(g) * u).astype(jnp.bfloat16)
        part = jnp.dot(h, wd_ref[0], preferred_element_type=jnp.float32)

        @pl.when(j == 0)
        def _():
            o_ref[rows, :] = part

        @pl.when(j > 0)
        def _():
            o_ref[rows, :] = o_ref[rows, :] + part

        @pl.when(j == NJ - 1)
        def _():
            o_ref[rows, :] = o_ref[rows, :] * tw_ref[rows, :]

        return 0

    jax.lax.fori_loop(0, nblk, blk, 0)


def _grouped_mlp(x_p, Wg, Wu, Wd, tw_p, sp):
    grid_spec = pltpu.PrefetchScalarGridSpec(
        num_scalar_prefetch=1,
        grid=(E, NJ),
        in_specs=[
            pl.BlockSpec((PADX, D), lambda e, j, sp: (0, 0)),
            pl.BlockSpec((1, D, BJ), lambda e, j, sp: (e, 0, j)),
            pl.BlockSpec((1, D, BJ), lambda e, j, sp: (e, 0, j)),
            pl.BlockSpec((1, BJ, D), lambda e, j, sp: (e, j, 0)),
            pl.BlockSpec((PADX, 1), lambda e, j, sp: (0, 0)),
        ],
        out_specs=pl.BlockSpec((PADX, D), lambda e, j, sp: (0, 0)),
    )
    return pl.pallas_call(
        _moe_body,
        grid_spec=grid_spec,
        out_shape=jax.ShapeDtypeStruct((PADX, D), jnp.float32),
    )(sp, x_p, Wg, Wu, Wd, tw_p)


def kernel(hidden_states, router_W, router_b, Wg, Wu, Wd):
    B, S, _ = hidden_states.shape
    flat = hidden_states.reshape(T, D)

    tw, ti = _router(flat, router_W, router_b)
    topi = ti[:, 0]
    topw = tw[:, 0]

    order = jnp.argsort(topi, stable=True).astype(jnp.int32)
    counts = jnp.bincount(topi, length=E)
    n8 = (counts + 7) // 8 * 8
    astart = jnp.cumsum(n8) - n8
    nblk = (counts + BT - 1) // BT
    cstart = jnp.cumsum(counts) - counts
    e_sorted = topi[order]
    pos = (astart[e_sorted] + jnp.arange(T) - cstart[e_sorted]).astype(jnp.int32)
    dest = jnp.zeros((T,), jnp.int32).at[order].set(pos)
    src = jnp.zeros((PADX,), jnp.int32).at[pos].set(order)
    sp = jnp.concatenate([astart, nblk]).astype(jnp.int32)

    x_p = flat.astype(jnp.bfloat16)[src]
    tw_p = topw[src].reshape(PADX, 1)

    y_p = _grouped_mlp(x_p, Wg.astype(jnp.bfloat16), Wu.astype(jnp.bfloat16),
                       Wd.astype(jnp.bfloat16), tw_p, sp)
    out = y_p[dest]
    return out.reshape(B, S, D)


# speedup vs baseline: 1.1042x; 1.0041x over previous
import functools

import jax
import jax.numpy as jnp
from jax.experimental import pallas as pl
from jax.experimental.pallas import tpu as pltpu
from jax.experimental.pallas import tpu_sc as plsc

D = 2048
F = 4096
E = 8
T = 2048
BT = 256
PADX = T + 8 * E + BT
BJ = 512
NJ = F // BJ


def _router_body(x_ref, rw_ref, rb_ref, tw_ref, ti_ref):
    l = jnp.dot(x_ref[...], rw_ref[...], preferred_element_type=jnp.float32)
    l = l + rb_ref[...]
    m = jnp.max(l, axis=1, keepdims=True)
    s = jnp.sum(jnp.exp(l - m), axis=1, keepdims=True)
    tw_ref[...] = 1.0 / s
    iota = jax.lax.broadcasted_iota(jnp.int32, l.shape, 1)
    ti_ref[...] = jnp.min(jnp.where(l >= m, iota, E), axis=1, keepdims=True)


def _router(flat, rw, rb):
    return pl.pallas_call(
        _router_body,
        out_shape=(
            jax.ShapeDtypeStruct((T, 1), jnp.float32),
            jax.ShapeDtypeStruct((T, 1), jnp.int32),
        ),
    )(flat, rw, rb.reshape(1, E))


def _moe_body(sp_ref, x_ref, wg_ref, wu_ref, wd_ref, tw_ref, o_ref):
    j = pl.program_id(1)
    e = pl.program_id(0)
    row0 = sp_ref[e]
    nblk = sp_ref[E + e]

    def blk(k, _):
        rows = pl.ds(pl.multiple_of(row0 + k * BT, 8), BT)
        x = x_ref[rows, :]
        g = jnp.dot(x, wg_ref[0], preferred_element_type=jnp.float32)
        u = jnp.dot(x, wu_ref[0], preferred_element_type=jnp.float32)
        h = (jax.nn.silu(g) * u).astype(jnp.bfloat16)
        part = jnp.dot(h, wd_ref[0], preferred_element_type=jnp.float32)

        @pl.when(j == 0)
        def _():
            o_ref[rows, :] = part

        @pl.when(j > 0)
        def _():
            o_ref[rows, :] = o_ref[rows, :] + part

        @pl.when(j == NJ - 1)
        def _():
            o_ref[rows, :] = o_ref[rows, :] * tw_ref[rows, :]

        return 0

    jax.lax.fori_loop(0, nblk, blk, 0)


def _grouped_mlp(x_p, Wg, Wu, Wd, tw_p, sp):
    grid_spec = pltpu.PrefetchScalarGridSpec(
        num_scalar_prefetch=1,
        grid=(E, NJ),
        in_specs=[
            pl.BlockSpec((PADX, D), lambda e, j, sp: (0, 0)),
            pl.BlockSpec((1, D, BJ), lambda e, j, sp: (e, 0, j)),
            pl.BlockSpec((1, D, BJ), lambda e, j, sp: (e, 0, j)),
            pl.BlockSpec((1, BJ, D), lambda e, j, sp: (e, j, 0)),
            pl.BlockSpec((PADX, 1), lambda e, j, sp: (0, 0)),
        ],
        out_specs=pl.BlockSpec((PADX, D), lambda e, j, sp: (0, 0)),
    )
    return pl.pallas_call(
        _moe_body,
        grid_spec=grid_spec,
        out_shape=jax.ShapeDtypeStruct((PADX, D), jnp.float32),
    )(sp, x_p, Wg, Wu, Wd, tw_p)


def _unpermute(y_p, dest):
    info = plsc.get_sparse_core_info()
    nc, ns = info.num_cores, info.num_subcores
    nw = nc * ns
    rows_per = T // nw
    ch = 32
    mesh = plsc.VectorSubcoreMesh(core_axis_name="c", subcore_axis_name="s")

    @functools.partial(
        pl.kernel, mesh=mesh,
        out_type=jax.ShapeDtypeStruct((T, D), jnp.float32),
        scratch_types=[
            pltpu.VMEM((ch,), jnp.int32),
            pltpu.VMEM((ch, D), jnp.float32),
            pltpu.SemaphoreType.DMA,
        ],
    )
    def gather_k(y_hbm, dest_hbm, out_hbm, idx_v, rows_v, sem):
        wid = jax.lax.axis_index("s") * nc + jax.lax.axis_index("c")
        base = wid * rows_per
        for c in range(rows_per // ch):
            off = base + c * ch
            pltpu.sync_copy(dest_hbm.at[pl.ds(off, ch)], idx_v)
            pltpu.async_copy(y_hbm.at[idx_v], rows_v, sem).wait()
            pltpu.sync_copy(rows_v, out_hbm.at[pl.ds(off, ch)])

    return gather_k(y_p, dest)


def kernel(hidden_states, router_W, router_b, Wg, Wu, Wd):
    B, S, _ = hidden_states.shape
    flat = hidden_states.reshape(T, D)

    tw, ti = _router(flat, router_W, router_b)
    topi = ti[:, 0]
    topw = tw[:, 0]

    order = jnp.argsort(topi, stable=True).astype(jnp.int32)
    counts = jnp.bincount(topi, length=E)
    n8 = (counts + 7) // 8 * 8
    astart = jnp.cumsum(n8) - n8
    nblk = (counts + BT - 1) // BT
    cstart = jnp.cumsum(counts) - counts
    e_sorted = topi[order]
    pos = (astart[e_sorted] + jnp.arange(T) - cstart[e_sorted]).astype(jnp.int32)
    dest = jnp.zeros((T,), jnp.int32).at[order].set(pos)
    src = jnp.zeros((PADX,), jnp.int32).at[pos].set(order)
    sp = jnp.concatenate([astart, nblk]).astype(jnp.int32)

    x_p = flat.astype(jnp.bfloat16)[src]
    tw_p = topw[src].reshape(PADX, 1)

    y_p = _grouped_mlp(x_p, Wg.astype(jnp.bfloat16), Wu.astype(jnp.bfloat16),
                       Wd.astype(jnp.bfloat16), tw_p, sp)
    out = _unpermute(y_p, dest)
    return out.reshape(B, S, D)
